# Initial kernel scaffold; baseline (speedup 1.0000x reference)
import jax
import jax.numpy as jnp
from jax import lax
from jax.experimental import pallas as pl
from jax.experimental.pallas import tpu as pltpu
from jax.experimental.pallas import tpu_sc as plsc

N_NODES = 10000
D = 128
N_EDGES = 320000

NPAD = 10240
NC, NS = 2, 16
NW = NC * NS
E_PER_TILE = N_EDGES // NW
K = 80
NCHUNK = E_PER_TILE // K
RPT = NPAD // NS

_mesh = plsc.VectorSubcoreMesh(core_axis_name="c", subcore_axis_name="s")


def _deg_body(dst_hbm, out_hbm, didx_v, deg_v, row_v, res_v, stage_sh):
    cid = lax.axis_index("c")
    sid = lax.axis_index("s")
    g = cid * NS + sid

    zeros16 = jnp.zeros((16,), jnp.float32)

    def zero_deg(i, carry):
        deg_v[pl.ds(i * 16, 16)] = zeros16
        return carry

    lax.fori_loop(0, NPAD // 16, zero_deg, 0)

    pltpu.sync_copy(dst_hbm.at[pl.ds(g * E_PER_TILE, E_PER_TILE)], didx_v)

    ones16 = jnp.ones((16,), jnp.float32)

    def acc_body(i, carry):
        idx = didx_v[pl.ds(i * 16, 16)]
        plsc.addupdate_scatter(deg_v, [idx], ones16)
        return carry

    lax.fori_loop(0, E_PER_TILE // 16, acc_body, 0)

    pltpu.sync_copy(deg_v, stage_sh.at[sid])
    plsc.subcore_barrier()

    def zero_res(i, carry):
        res_v[pl.ds(i * 16, 16)] = zeros16
        return carry

    lax.fori_loop(0, RPT // 16, zero_res, 0)

    for r in range(NS):
        pltpu.sync_copy(stage_sh.at[r, pl.ds(sid * RPT, RPT)], row_v)

        def add_body(ci, carry):
            sl = pl.ds(ci * 16, 16)
            res_v[sl] = res_v[sl] + row_v[sl]
            return carry

        lax.fori_loop(0, RPT // 16, add_body, 0)

    pltpu.sync_copy(res_v, out_hbm.at[cid, pl.ds(sid * RPT, RPT)])


_deg_kernel = pl.kernel(
    _deg_body,
    out_type=jax.ShapeDtypeStruct((NC, NPAD), jnp.float32),
    mesh=_mesh,
    scratch_types=[
        pltpu.VMEM((E_PER_TILE,), jnp.int32),
        pltpu.VMEM((NPAD,), jnp.float32),
        pltpu.VMEM((RPT,), jnp.float32),
        pltpu.VMEM((RPT,), jnp.float32),
        pltpu.VMEM_SHARED((NS, NPAD), jnp.float32),
    ],
)


def _agg_body(y_hbm, src_hbm, dst_hbm, out_hbm, sidx_v, didx_v, rows_v, acc_sh):
    cid = lax.axis_index("c")
    sid = lax.axis_index("s")
    g = cid * NS + sid
    rbase = sid * RPT

    zeros16 = jnp.zeros((16,), jnp.float32)

    @pl.when(cid == 0)
    def _():
        for b in range(RPT // K):
            sl = pl.ds(rbase + b * K, K)
            pltpu.sync_copy(y_hbm.at[sl], rows_v)
            pltpu.sync_copy(rows_v, acc_sh.at[sl])

    @pl.when(cid != 0)
    def _():
        def zero_rows(i, carry):
            for j in range(D // 16):
                rows_v[i, pl.ds(j * 16, 16)] = zeros16
            return carry

        lax.fori_loop(0, K, zero_rows, 0)
        for b in range(RPT // K):
            pltpu.sync_copy(rows_v, acc_sh.at[pl.ds(rbase + b * K, K)])

    plsc.subcore_barrier()

    ebase = g * E_PER_TILE

    def edge_body(i, carry):
        off = ebase + i * K
        pltpu.sync_copy(src_hbm.at[pl.ds(off, K)], sidx_v)
        pltpu.sync_copy(y_hbm.at[sidx_v], rows_v)
        pltpu.sync_copy(dst_hbm.at[pl.ds(off, K)], didx_v)
        pltpu.sync_copy(rows_v, acc_sh.at[didx_v], add=True)
        return carry

    lax.fori_loop(0, NCHUNK, edge_body, 0)
    plsc.subcore_barrier()

    for b in range(RPT // K):
        sl = pl.ds(rbase + b * K, K)
        pltpu.sync_copy(acc_sh.at[sl], rows_v)
        pltpu.sync_copy(rows_v, out_hbm.at[cid, sl])


_agg_kernel = pl.kernel(
    _agg_body,
    out_type=jax.ShapeDtypeStruct((NC, NPAD, D), jnp.float32),
    mesh=_mesh,
    scratch_types=[
        pltpu.VMEM((K,), jnp.int32),
        pltpu.VMEM((K,), jnp.int32),
        pltpu.VMEM((K, D), jnp.float32),
        pltpu.VMEM_SHARED((NPAD, D), jnp.float32),
    ],
)


BLK = 1024
GRID = NPAD // BLK


def _mm_body(x_ref, w_ref, o_ref):
    o_ref[...] = jnp.dot(x_ref[...], w_ref[...], preferred_element_type=jnp.float32)


def _tc_matmul(x, w):
    return pl.pallas_call(
        _mm_body,
        grid=(GRID,),
        in_specs=[
            pl.BlockSpec((BLK, D), lambda i: (i, 0)),
            pl.BlockSpec((D, D), lambda i: (0, 0)),
        ],
        out_specs=pl.BlockSpec((BLK, D), lambda i: (i, 0)),
        out_shape=jax.ShapeDtypeStruct((NPAD, D), jnp.float32),
    )(x, w)


def _scale_body(degT_ref, xw_ref, y_ref, dinv_ref):
    d = degT_ref[...]
    dinv = lax.rsqrt(d[:, 0:1] + d[:, 1:2] + 1.0)
    dinv_ref[...] = dinv
    y_ref[...] = xw_ref[...] * dinv


def _tc_scale(degT, xw):
    return pl.pallas_call(
        _scale_body,
        grid=(GRID,),
        in_specs=[
            pl.BlockSpec((BLK, 2), lambda i: (i, 0)),
            pl.BlockSpec((BLK, D), lambda i: (i, 0)),
        ],
        out_specs=[
            pl.BlockSpec((BLK, D), lambda i: (i, 0)),
            pl.BlockSpec((BLK, 1), lambda i: (i, 0)),
        ],
        out_shape=[
            jax.ShapeDtypeStruct((NPAD, D), jnp.float32),
            jax.ShapeDtypeStruct((NPAD, 1), jnp.float32),
        ],
    )(degT, xw)


def _mid_body(p0_ref, p1_ref, dinv_ref, b1_ref, w2_ref, y2_ref):
    dinv = dinv_ref[...]
    h = jnp.maximum((p0_ref[...] + p1_ref[...]) * dinv + b1_ref[...], 0.0)
    y2_ref[...] = jnp.dot(h, w2_ref[...], preferred_element_type=jnp.float32) * dinv


def _tc_mid(p0, p1, dinv, b1, w2):
    return pl.pallas_call(
        _mid_body,
        grid=(GRID,),
        in_specs=[
            pl.BlockSpec((BLK, D), lambda i: (i, 0)),
            pl.BlockSpec((BLK, D), lambda i: (i, 0)),
            pl.BlockSpec((BLK, 1), lambda i: (i, 0)),
            pl.BlockSpec((1, D), lambda i: (0, 0)),
            pl.BlockSpec((D, D), lambda i: (0, 0)),
        ],
        out_specs=pl.BlockSpec((BLK, D), lambda i: (i, 0)),
        out_shape=jax.ShapeDtypeStruct((NPAD, D), jnp.float32),
    )(p0, p1, dinv, b1, w2)


def _out_body(q0_ref, q1_ref, dinv_ref, b2_ref, o_ref):
    o_ref[...] = (q0_ref[...] + q1_ref[...]) * dinv_ref[...] + b2_ref[...]


def _tc_out(q0, q1, dinv, b2):
    return pl.pallas_call(
        _out_body,
        grid=(GRID,),
        in_specs=[
            pl.BlockSpec((BLK, D), lambda i: (i, 0)),
            pl.BlockSpec((BLK, D), lambda i: (i, 0)),
            pl.BlockSpec((BLK, 1), lambda i: (i, 0)),
            pl.BlockSpec((1, D), lambda i: (0, 0)),
        ],
        out_specs=pl.BlockSpec((BLK, D), lambda i: (i, 0)),
        out_shape=jax.ShapeDtypeStruct((NPAD, D), jnp.float32),
    )(q0, q1, dinv, b2)


def kernel(x, edge_index, W1, b1, W2, b2):
    src = edge_index[0].astype(jnp.int32)
    dst = edge_index[1].astype(jnp.int32)
    xp = jnp.pad(x, ((0, NPAD - N_NODES), (0, 0)))

    deg = _deg_kernel(dst)
    xw1 = _tc_matmul(xp, W1)
    y1, dinv = _tc_scale(deg.T, xw1)

    p = _agg_kernel(y1, src, dst)
    y2 = _tc_mid(p[0], p[1], dinv, b1.reshape(1, D), W2)

    q = _agg_kernel(y2, src, dst)
    out = _tc_out(q[0], q[1], dinv, b2.reshape(1, D))
    return out[:N_NODES]


# speedup vs baseline: 13.5422x; 13.5422x over previous
import jax
import jax.numpy as jnp
from jax import lax
from jax.experimental import pallas as pl
from jax.experimental.pallas import tpu as pltpu
from jax.experimental.pallas import tpu_sc as plsc

N_NODES = 10000
D = 128
N_EDGES = 320000

NPAD = 10240
NC, NS = 2, 16
NW = NC * NS
E_PER_TILE = N_EDGES // NW
K = 80
NCHUNK = E_PER_TILE // K
RPT = NPAD // NS

_mesh = plsc.VectorSubcoreMesh(core_axis_name="c", subcore_axis_name="s")


def _deg_body(dst_hbm, out_hbm, didx_v, deg_v, row_v, res_v, stage_sh):
    cid = lax.axis_index("c")
    sid = lax.axis_index("s")
    g = cid * NS + sid

    zeros16 = jnp.zeros((16,), jnp.float32)

    def zero_deg(i, carry):
        deg_v[pl.ds(i * 16, 16)] = zeros16
        return carry

    lax.fori_loop(0, NPAD // 16, zero_deg, 0)

    pltpu.sync_copy(dst_hbm.at[pl.ds(g * E_PER_TILE, E_PER_TILE)], didx_v)

    ones16 = jnp.ones((16,), jnp.float32)

    def acc_body(i, carry):
        idx = didx_v[pl.ds(i * 16, 16)]
        plsc.addupdate_scatter(deg_v, [idx], ones16)
        return carry

    lax.fori_loop(0, E_PER_TILE // 16, acc_body, 0)

    pltpu.sync_copy(deg_v, stage_sh.at[sid])
    plsc.subcore_barrier()

    def zero_res(i, carry):
        res_v[pl.ds(i * 16, 16)] = zeros16
        return carry

    lax.fori_loop(0, RPT // 16, zero_res, 0)

    for r in range(NS):
        pltpu.sync_copy(stage_sh.at[r, pl.ds(sid * RPT, RPT)], row_v)

        def add_body(ci, carry):
            sl = pl.ds(ci * 16, 16)
            res_v[sl] = res_v[sl] + row_v[sl]
            return carry

        lax.fori_loop(0, RPT // 16, add_body, 0)

    pltpu.sync_copy(res_v, out_hbm.at[cid, pl.ds(sid * RPT, RPT)])


_deg_kernel = pl.kernel(
    _deg_body,
    out_type=jax.ShapeDtypeStruct((NC, NPAD), jnp.float32),
    mesh=_mesh,
    scratch_types=[
        pltpu.VMEM((E_PER_TILE,), jnp.int32),
        pltpu.VMEM((NPAD,), jnp.float32),
        pltpu.VMEM((RPT,), jnp.float32),
        pltpu.VMEM((RPT,), jnp.float32),
        pltpu.VMEM_SHARED((NS, NPAD), jnp.float32),
    ],
    compiler_params=pltpu.CompilerParams(needs_layout_passes=False),
)


def _agg_body(y_hbm, src_hbm, dst_hbm, out_hbm, sidx_v, didx_v, rows_v, acc_sh):
    cid = lax.axis_index("c")
    sid = lax.axis_index("s")
    g = cid * NS + sid
    rbase = sid * RPT

    zeros16 = jnp.zeros((16,), jnp.float32)

    @pl.when(cid == 0)
    def _():
        for b in range(RPT // K):
            sl = pl.ds(rbase + b * K, K)
            pltpu.sync_copy(y_hbm.at[sl], rows_v)
            pltpu.sync_copy(rows_v, acc_sh.at[sl])

    @pl.when(cid != 0)
    def _():
        def zero_rows(i, carry):
            for j in range(D // 16):
                rows_v[i, pl.ds(j * 16, 16)] = zeros16
            return carry

        lax.fori_loop(0, K, zero_rows, 0)
        for b in range(RPT // K):
            pltpu.sync_copy(rows_v, acc_sh.at[pl.ds(rbase + b * K, K)])

    plsc.subcore_barrier()

    ebase = g * E_PER_TILE

    def edge_body(i, carry):
        off = ebase + i * K
        pltpu.sync_copy(src_hbm.at[pl.ds(off, K)], sidx_v)
        pltpu.sync_copy(y_hbm.at[sidx_v], rows_v)
        pltpu.sync_copy(dst_hbm.at[pl.ds(off, K)], didx_v)
        pltpu.sync_copy(rows_v, acc_sh.at[didx_v], add=True)
        return carry

    lax.fori_loop(0, NCHUNK, edge_body, 0)
    plsc.subcore_barrier()

    for b in range(RPT // K):
        sl = pl.ds(rbase + b * K, K)
        pltpu.sync_copy(acc_sh.at[sl], rows_v)
        pltpu.sync_copy(rows_v, out_hbm.at[cid, sl])


_agg_kernel = pl.kernel(
    _agg_body,
    out_type=jax.ShapeDtypeStruct((NC, NPAD, D), jnp.float32),
    mesh=_mesh,
    scratch_types=[
        pltpu.VMEM((K,), jnp.int32),
        pltpu.VMEM((K,), jnp.int32),
        pltpu.VMEM((K, D), jnp.float32),
        pltpu.VMEM_SHARED((NPAD, D), jnp.float32),
    ],
)


BLK = 1024
GRID = NPAD // BLK


def _mm_body(x_ref, w_ref, o_ref):
    o_ref[...] = jnp.dot(x_ref[...], w_ref[...], preferred_element_type=jnp.float32)


def _tc_matmul(x, w):
    return pl.pallas_call(
        _mm_body,
        grid=(GRID,),
        in_specs=[
            pl.BlockSpec((BLK, D), lambda i: (i, 0)),
            pl.BlockSpec((D, D), lambda i: (0, 0)),
        ],
        out_specs=pl.BlockSpec((BLK, D), lambda i: (i, 0)),
        out_shape=jax.ShapeDtypeStruct((NPAD, D), jnp.float32),
    )(x, w)


def _scale_body(degT_ref, xw_ref, y_ref, dinv_ref):
    d = degT_ref[...]
    dinv = lax.rsqrt(d[:, 0:1] + d[:, 1:2] + 1.0)
    dinv_ref[...] = dinv
    y_ref[...] = xw_ref[...] * dinv


def _tc_scale(degT, xw):
    return pl.pallas_call(
        _scale_body,
        grid=(GRID,),
        in_specs=[
            pl.BlockSpec((BLK, 2), lambda i: (i, 0)),
            pl.BlockSpec((BLK, D), lambda i: (i, 0)),
        ],
        out_specs=[
            pl.BlockSpec((BLK, D), lambda i: (i, 0)),
            pl.BlockSpec((BLK, 1), lambda i: (i, 0)),
        ],
        out_shape=[
            jax.ShapeDtypeStruct((NPAD, D), jnp.float32),
            jax.ShapeDtypeStruct((NPAD, 1), jnp.float32),
        ],
    )(degT, xw)


def _mid_body(p0_ref, p1_ref, dinv_ref, b1_ref, w2_ref, y2_ref):
    dinv = dinv_ref[...]
    h = jnp.maximum((p0_ref[...] + p1_ref[...]) * dinv + b1_ref[...], 0.0)
    y2_ref[...] = jnp.dot(h, w2_ref[...], preferred_element_type=jnp.float32) * dinv


def _tc_mid(p0, p1, dinv, b1, w2):
    return pl.pallas_call(
        _mid_body,
        grid=(GRID,),
        in_specs=[
            pl.BlockSpec((BLK, D), lambda i: (i, 0)),
            pl.BlockSpec((BLK, D), lambda i: (i, 0)),
            pl.BlockSpec((BLK, 1), lambda i: (i, 0)),
            pl.BlockSpec((1, D), lambda i: (0, 0)),
            pl.BlockSpec((D, D), lambda i: (0, 0)),
        ],
        out_specs=pl.BlockSpec((BLK, D), lambda i: (i, 0)),
        out_shape=jax.ShapeDtypeStruct((NPAD, D), jnp.float32),
    )(p0, p1, dinv, b1, w2)


def _out_body(q0_ref, q1_ref, dinv_ref, b2_ref, o_ref):
    o_ref[...] = (q0_ref[...] + q1_ref[...]) * dinv_ref[...] + b2_ref[...]


def _tc_out(q0, q1, dinv, b2):
    return pl.pallas_call(
        _out_body,
        grid=(GRID,),
        in_specs=[
            pl.BlockSpec((BLK, D), lambda i: (i, 0)),
            pl.BlockSpec((BLK, D), lambda i: (i, 0)),
            pl.BlockSpec((BLK, 1), lambda i: (i, 0)),
            pl.BlockSpec((1, D), lambda i: (0, 0)),
        ],
        out_specs=pl.BlockSpec((BLK, D), lambda i: (i, 0)),
        out_shape=jax.ShapeDtypeStruct((NPAD, D), jnp.float32),
    )(q0, q1, dinv, b2)


def kernel(x, edge_index, W1, b1, W2, b2):
    src = edge_index[0].astype(jnp.int32)
    dst = edge_index[1].astype(jnp.int32)
    xp = jnp.pad(x, ((0, NPAD - N_NODES), (0, 0)))

    deg = _deg_kernel(dst)
    xw1 = _tc_matmul(xp, W1)
    y1, dinv = _tc_scale(deg.T, xw1)

    p = _agg_kernel(y1, src, dst)
    y2 = _tc_mid(p[0], p[1], dinv, b1.reshape(1, D), W2)

    q = _agg_kernel(y2, src, dst)
    out = _tc_out(q[0], q[1], dinv, b2.reshape(1, D))
    return out[:N_NODES]
